# Initial kernel scaffold; baseline (speedup 1.0000x reference)
import jax
import jax.numpy as jnp
from jax.experimental import pallas as pl


def kernel(x, edge_index, pre_W1, pre_b1, post_W1, post_b1, lin_W1, lin_b1, pre_W2, pre_b2, post_W2, post_b2, lin_W2, lin_b2, bn1_gamma, bn1_beta, bn2_gamma, bn2_beta):
    raise NotImplementedError("write your pallas kernel here")


# speedup vs baseline: 1.9098x; 1.9098x over previous
import functools
import math

import jax
import jax.numpy as jnp
import numpy as np
from jax import lax
from jax.experimental import pallas as pl
from jax.experimental.pallas import tpu as pltpu
from jax.experimental.pallas import tpu_sc as plsc

N = 10000
D = 128
E = 320000

_DEG_HIST = np.array([0] * 32 + [10000], dtype=np.float64)
_BINS = np.arange(_DEG_HIST.shape[0], dtype=np.float64)
_AVG_DEG_LOG = float((np.log(_BINS + 1.0) * _DEG_HIST).sum() / _DEG_HIST.sum())

_ROW_BLK = 2000
_GRID = N // _ROW_BLK


def _pre_body(x_ref, wd_ref, ws_ref, b_ref, a_ref, bt_ref, b2_ref):
    x = x_ref[...]
    a_ref[...] = jnp.dot(x, wd_ref[...], preferred_element_type=jnp.float32) + b_ref[...]
    bt = jnp.dot(x, ws_ref[...], preferred_element_type=jnp.float32)
    bt_ref[...] = bt
    b2_ref[...] = bt * bt


def _tc_pre(x, wd, ws, b):
    return pl.pallas_call(
        _pre_body,
        grid=(_GRID,),
        in_specs=[
            pl.BlockSpec((_ROW_BLK, D), lambda i: (i, 0)),
            pl.BlockSpec((D, D), lambda i: (0, 0)),
            pl.BlockSpec((D, D), lambda i: (0, 0)),
            pl.BlockSpec((1, D), lambda i: (0, 0)),
        ],
        out_specs=[
            pl.BlockSpec((_ROW_BLK, D), lambda i: (i, 0)),
            pl.BlockSpec((_ROW_BLK, D), lambda i: (i, 0)),
            pl.BlockSpec((_ROW_BLK, D), lambda i: (i, 0)),
        ],
        out_shape=[jax.ShapeDtypeStruct((N, D), jnp.float32)] * 3,
    )(x, wd, ws, b.reshape(1, D))


def _bnpre_body(h_ref, s_ref, sq_ref, g_ref, be_ref, wd_ref, ws_ref, b_ref,
                hn_ref, a_ref, bt_ref, b2_ref):
    mu = s_ref[...] / N
    var = sq_ref[...] / N - mu * mu
    inv = lax.rsqrt(var + 1e-5) * g_ref[...]
    hn = jnp.maximum((h_ref[...] - mu) * inv + be_ref[...], 0.0)
    hn_ref[...] = hn
    a_ref[...] = jnp.dot(hn, wd_ref[...], preferred_element_type=jnp.float32) + b_ref[...]
    bt = jnp.dot(hn, ws_ref[...], preferred_element_type=jnp.float32)
    bt_ref[...] = bt
    b2_ref[...] = bt * bt


def _tc_bn_pre(h_raw, s, sq, gamma, beta, wd, ws, b):
    return pl.pallas_call(
        _bnpre_body,
        grid=(_GRID,),
        in_specs=[
            pl.BlockSpec((_ROW_BLK, D), lambda i: (i, 0)),
            pl.BlockSpec((1, D), lambda i: (0, 0)),
            pl.BlockSpec((1, D), lambda i: (0, 0)),
            pl.BlockSpec((1, D), lambda i: (0, 0)),
            pl.BlockSpec((1, D), lambda i: (0, 0)),
            pl.BlockSpec((D, D), lambda i: (0, 0)),
            pl.BlockSpec((D, D), lambda i: (0, 0)),
            pl.BlockSpec((1, D), lambda i: (0, 0)),
        ],
        out_specs=[pl.BlockSpec((_ROW_BLK, D), lambda i: (i, 0))] * 4,
        out_shape=[jax.ShapeDtypeStruct((N, D), jnp.float32)] * 4,
    )(h_raw, s, sq, gamma.reshape(1, D), beta.reshape(1, D), wd, ws, b.reshape(1, D))


def _post_body(x_ref, a_ref, s_ref, sq_ref, mn_ref, mx_ref, cnt_ref,
               pw_ref, pb_ref, lw_ref, lb_ref,
               y_ref, ps_ref, psq_ref):
    i = pl.program_id(0)
    count = cnt_ref[:, 0:1]
    cnt = jnp.maximum(count, 1.0)
    has = count > 0.0
    a = a_ref[...]
    s = s_ref[...]
    mean = (count * a + s) / cnt
    mn = jnp.where(has, a + mn_ref[...], 0.0)
    mx = jnp.where(has, a + mx_ref[...], 0.0)
    sc = s / cnt
    std = jnp.sqrt(jnp.maximum(sq_ref[...] / cnt - sc * sc, 0.0) + 1e-5)
    aggr = jnp.concatenate([mean, mn, mx, std], axis=-1)
    lg = jnp.log(cnt + 1.0)
    amp = lg * (1.0 / _AVG_DEG_LOG)
    att = _AVG_DEG_LOG / lg
    p0 = pw_ref[0:D, :]
    p1 = pw_ref[D:5 * D, :]
    p2 = pw_ref[5 * D:9 * D, :]
    p3 = pw_ref[9 * D:13 * D, :]
    y = (jnp.dot(x_ref[...], p0, preferred_element_type=jnp.float32)
         + jnp.dot(aggr, p1, preferred_element_type=jnp.float32)
         + amp * jnp.dot(aggr, p2, preferred_element_type=jnp.float32)
         + att * jnp.dot(aggr, p3, preferred_element_type=jnp.float32)
         + pb_ref[...])
    y = jnp.dot(y, lw_ref[...], preferred_element_type=jnp.float32) + lb_ref[...]
    y_ref[...] = y

    @pl.when(i == 0)
    def _():
        ps_ref[...] = jnp.zeros_like(ps_ref)
        psq_ref[...] = jnp.zeros_like(psq_ref)

    ps_ref[...] += jnp.sum(y, axis=0, keepdims=True)
    psq_ref[...] += jnp.sum(y * y, axis=0, keepdims=True)


def _tc_post(x, a, s, sq, mn, mx, cnt16, post_w, post_b, lin_w, lin_b):
    return pl.pallas_call(
        _post_body,
        grid=(_GRID,),
        in_specs=[
            pl.BlockSpec((_ROW_BLK, D), lambda i: (i, 0)),
            pl.BlockSpec((_ROW_BLK, D), lambda i: (i, 0)),
            pl.BlockSpec((_ROW_BLK, D), lambda i: (i, 0)),
            pl.BlockSpec((_ROW_BLK, D), lambda i: (i, 0)),
            pl.BlockSpec((_ROW_BLK, D), lambda i: (i, 0)),
            pl.BlockSpec((_ROW_BLK, D), lambda i: (i, 0)),
            pl.BlockSpec((_ROW_BLK, 16), lambda i: (i, 0)),
            pl.BlockSpec((13 * D, D), lambda i: (0, 0)),
            pl.BlockSpec((1, D), lambda i: (0, 0)),
            pl.BlockSpec((D, D), lambda i: (0, 0)),
            pl.BlockSpec((1, D), lambda i: (0, 0)),
        ],
        out_specs=[
            pl.BlockSpec((_ROW_BLK, D), lambda i: (i, 0)),
            pl.BlockSpec((1, D), lambda i: (0, 0)),
            pl.BlockSpec((1, D), lambda i: (0, 0)),
        ],
        out_shape=[
            jax.ShapeDtypeStruct((N, D), jnp.float32),
            jax.ShapeDtypeStruct((1, D), jnp.float32),
            jax.ShapeDtypeStruct((1, D), jnp.float32),
        ],
    )(x, a, s, sq, mn, mx, cnt16, post_w, post_b.reshape(1, D),
      lin_w, lin_b.reshape(1, D))


def _final_body(h_ref, s_ref, sq_ref, g_ref, be_ref, x_ref, o_ref):
    mu = s_ref[...] / N
    var = sq_ref[...] / N - mu * mu
    inv = lax.rsqrt(var + 1e-5) * g_ref[...]
    hn = jnp.maximum((h_ref[...] - mu) * inv + be_ref[...], 0.0)
    o_ref[...] = (hn + x_ref[...]) * np.float32(1.0 / math.sqrt(2.0))


def _tc_final(h_raw, s, sq, gamma, beta, x):
    return pl.pallas_call(
        _final_body,
        grid=(_GRID,),
        in_specs=[
            pl.BlockSpec((_ROW_BLK, D), lambda i: (i, 0)),
            pl.BlockSpec((1, D), lambda i: (0, 0)),
            pl.BlockSpec((1, D), lambda i: (0, 0)),
            pl.BlockSpec((1, D), lambda i: (0, 0)),
            pl.BlockSpec((1, D), lambda i: (0, 0)),
            pl.BlockSpec((_ROW_BLK, D), lambda i: (i, 0)),
        ],
        out_specs=pl.BlockSpec((_ROW_BLK, D), lambda i: (i, 0)),
        out_shape=jax.ShapeDtypeStruct((N, D), jnp.float32),
    )(h_raw, s, sq, gamma.reshape(1, D), beta.reshape(1, D), x)


_NT = 32
_NPT = 320
_NPAD = _NT * _NPT
_TRASH = _NPT
_C1 = 80
_EPT = E // 16
_C2 = 4000
_GB = 32
_NBMAX = _C2 // _GB + 1


def _sc_p1_body(tb, srch, dsth, ssq_out, stage, dstb, srcb, sacc):
    c = lax.axis_index("c")
    s = lax.axis_index("s")

    fzero = jnp.zeros((16,), jnp.float32)

    def _zstage(i, carry):
        stage[i >> 3, pl.ds((i & 7) * 16, 16)] = fzero
        return carry
    lax.fori_loop(0, 104 * 8, _zstage, 0)

    base = s * 624
    def _zsp(j, carry):
        pltpu.sync_copy(stage.at[pl.ds(0, 104)], sacc.at[pl.ds(base + j * 104, 104)])
        return carry
    lax.fori_loop(0, 6, _zsp, 0)

    @pl.when(s == 0)
    def _():
        pltpu.sync_copy(stage.at[pl.ds(0, 16)], sacc.at[pl.ds(9984, 16)])

    plsc.subcore_barrier()

    cN = c * N
    ebase = s * _EPT
    def _p1(ch, carry):
        e0 = ebase + ch * _C1
        pltpu.sync_copy(dsth.at[pl.ds(e0, _C1)], dstb)
        pltpu.sync_copy(srch.at[pl.ds(e0, _C1)], srcb)
        for j in range(_C1 // 16):
            srcb[pl.ds(j * 16, 16)] = srcb[pl.ds(j * 16, 16)] + cN
        stg = stage.at[pl.ds(0, _C1)]
        pltpu.sync_copy(tb.at[srcb], stg)
        pltpu.sync_copy(stg, sacc.at[dstb], add=True)
        return carry
    lax.fori_loop(0, _EPT // _C1, _p1, 0)

    plsc.subcore_barrier()

    ob = cN + base
    pltpu.sync_copy(sacc.at[pl.ds(base, 624)], ssq_out.at[pl.ds(ob, 624)])

    @pl.when(s == 0)
    def _():
        pltpu.sync_copy(sacc.at[pl.ds(9984, 16)], ssq_out.at[pl.ds(cN + 9984, 16)])


def _sc_p2_body(bt, srch, dsth, cnt_out, mn_out, mx_out,
                dchunk, schunk, mbuf,
                gidx, relb, brows, accmn, accmx, ccnt, cstage):
    c = lax.axis_index("c")
    s = lax.axis_index("s")
    wid = c * 16 + s
    i32 = jnp.int32
    fzero = jnp.zeros((16,), jnp.float32)
    fone = jnp.ones((16,), jnp.float32)
    zi = jnp.zeros((16,), i32)

    fmaxv = jnp.full((16,), 3.4028235e38, jnp.float32)
    fminv = -fmaxv

    def _iacc(r, carry):
        for k in range(8):
            accmn[r, pl.ds(k * 16, 16)] = fmaxv
            accmx[r, pl.ds(k * 16, 16)] = fminv
        ccnt[pl.ds(r * 16, 16)] = fzero
        return carry
    lax.fori_loop(0, _NPT + 1, _iacc, 0)

    lo = wid * _NPT
    trashpack = zi + (_NPT * 16384)

    def _p2(ch, carry):
        pltpu.sync_copy(dsth.at[pl.ds(ch * _C2, _C2)], dchunk)
        pltpu.sync_copy(srch.at[pl.ds(ch * _C2, _C2)], schunk)

        def _scan(g, mcnt):
            dv = dchunk[pl.ds(g * 16, 16)]
            sv = schunk[pl.ds(g * 16, 16)]
            rel = dv - lo
            m = (rel >= 0) & (rel < _NPT)
            mi = jnp.where(m, 1, 0)
            pv = jnp.where(m, rel * 16384 + sv, _NPT * 16384)
            for ln in range(16):
                mbuf[pl.ds(mcnt, 16)] = zi + pv[ln]
                mcnt = mcnt + mi[ln]
            return mcnt
        mcnt = lax.fori_loop(0, _C2 // 16, _scan, jnp.int32(0))

        mbuf[pl.ds(mcnt, 16)] = trashpack
        mbuf[pl.ds(mcnt + 16, 16)] = trashpack
        nb = lax.shift_right_logical(mcnt + (_GB - 1), 5)

        def _batch(b, carry2):
            @pl.when(b * _GB < mcnt)
            def _():
                for grp in range(2):
                    pvv = mbuf[pl.ds(b * _GB + grp * 16, 16)]
                    gidx[pl.ds(grp * 16, 16)] = pvv & 16383
                    relb[pl.ds(grp * 16, 16)] = lax.shift_right_arithmetic(pvv, 14)
                pltpu.sync_copy(bt.at[gidx], brows)
                for grp in range(2):
                    relv = relb[pl.ds(grp * 16, 16)] + 0
                    for ln in range(16):
                        r = relv[ln]
                        l = grp * 16 + ln
                        ccnt[pl.ds(r * 16, 16)] = ccnt[pl.ds(r * 16, 16)] + fone
                        for k in range(8):
                            seg = brows[l, pl.ds(k * 16, 16)]
                            accmn[r, pl.ds(k * 16, 16)] = jnp.minimum(
                                accmn[r, pl.ds(k * 16, 16)], seg)
                            accmx[r, pl.ds(k * 16, 16)] = jnp.maximum(
                                accmx[r, pl.ds(k * 16, 16)], seg)
            return carry2
        lax.fori_loop(0, _NBMAX, _batch, 0)
        return carry
    lax.fori_loop(0, E // _C2, _p2, 0)

    pltpu.sync_copy(accmn.at[pl.ds(0, _NPT)], mn_out.at[pl.ds(lo, _NPT)])
    pltpu.sync_copy(accmx.at[pl.ds(0, _NPT)], mx_out.at[pl.ds(lo, _NPT)])

    def _rp(j, carry):
        cstage[j >> 3, pl.ds((j & 7) * 16, 16)] = ccnt[pl.ds(j * 16, 16)] + 0.0
        return carry
    lax.fori_loop(0, _NPT, _rp, 0)
    pltpu.sync_copy(cstage.at[pl.ds(0, 40)], cnt_out.at[pl.ds(wid * 40, 40)])


def _sc_segment_stats(bt, b2, src, dst):
    mesh = plsc.VectorSubcoreMesh(core_axis_name="c", subcore_axis_name="s")
    f32 = jnp.float32
    k1 = pl.kernel(
        _sc_p1_body,
        out_type=[
            jax.ShapeDtypeStruct((2 * N, D), f32),
        ],
        mesh=mesh,
        scratch_types=[
            pltpu.VMEM((104, D), f32),
            pltpu.VMEM((_C1,), jnp.int32),
            pltpu.VMEM((_C1,), jnp.int32),
            pltpu.VMEM_SHARED((N, D), f32),
        ],
    )
    tb = jnp.concatenate([bt, b2], axis=0)
    ssq = k1(tb, src, dst)[0]
    s, sq = ssq[:N], ssq[N:]
    k2 = pl.kernel(
        _sc_p2_body,
        out_type=[
            jax.ShapeDtypeStruct((_NPAD // 8, D), f32),
            jax.ShapeDtypeStruct((_NPAD, D), f32),
            jax.ShapeDtypeStruct((_NPAD, D), f32),
        ],
        mesh=mesh,
        scratch_types=[
            pltpu.VMEM((_C2,), jnp.int32),
            pltpu.VMEM((_C2,), jnp.int32),
            pltpu.VMEM((_C2 + 48,), jnp.int32),
            pltpu.VMEM((32,), jnp.int32),
            pltpu.VMEM((32,), jnp.int32),
            pltpu.VMEM((32, D), f32),
            pltpu.VMEM((_NPT + 1, D), f32),
            pltpu.VMEM((_NPT + 1, D), f32),
            pltpu.VMEM(((_NPT + 1) * 16,), f32),
            pltpu.VMEM((40, D), f32),
        ],
    )
    cntp, mn, mx = k2(bt, src, dst)
    cnt16 = cntp.reshape(_NPAD, 16)[:N]
    return s, sq, mn[:N], mx[:N], cnt16


def kernel(x, edge_index, pre_W1, pre_b1, post_W1, post_b1, lin_W1, lin_b1,
           pre_W2, pre_b2, post_W2, post_b2, lin_W2, lin_b2,
           bn1_gamma, bn1_beta, bn2_gamma, bn2_beta):
    src = edge_index[0]
    dst = edge_index[1]

    a1, bt1, b21 = _tc_pre(x, pre_W1[:D], pre_W1[D:], pre_b1)
    s1, sq1, mn1, mx1, cnt16 = _sc_segment_stats(bt1, b21, src, dst)
    h1_raw, cs1, csq1 = _tc_post(x, a1, s1, sq1, mn1, mx1, cnt16,
                                 post_W1, post_b1, lin_W1, lin_b1)

    h1, a2, bt2, b22 = _tc_bn_pre(h1_raw, cs1, csq1, bn1_gamma, bn1_beta,
                                  pre_W2[:D], pre_W2[D:], pre_b2)
    s2, sq2, mn2, mx2, _ = _sc_segment_stats(bt2, b22, src, dst)
    h2_raw, cs2, csq2 = _tc_post(h1, a2, s2, sq2, mn2, mx2, cnt16,
                                 post_W2, post_b2, lin_W2, lin_b2)

    return _tc_final(h2_raw, cs2, csq2, bn2_gamma, bn2_beta, x)


# speedup vs baseline: 1.9512x; 1.0217x over previous
import functools
import math

import jax
import jax.numpy as jnp
import numpy as np
from jax import lax
from jax.experimental import pallas as pl
from jax.experimental.pallas import tpu as pltpu
from jax.experimental.pallas import tpu_sc as plsc

N = 10000
D = 128
E = 320000

_DEG_HIST = np.array([0] * 32 + [10000], dtype=np.float64)
_BINS = np.arange(_DEG_HIST.shape[0], dtype=np.float64)
_AVG_DEG_LOG = float((np.log(_BINS + 1.0) * _DEG_HIST).sum() / _DEG_HIST.sum())

_ROW_BLK = 2000
_GRID = N // _ROW_BLK


def _pre_body(x_ref, wd_ref, ws_ref, b_ref, a_ref, bt_ref, b2_ref):
    x = x_ref[...]
    a_ref[...] = jnp.dot(x, wd_ref[...], preferred_element_type=jnp.float32) + b_ref[...]
    bt = jnp.dot(x, ws_ref[...], preferred_element_type=jnp.float32)
    bt_ref[...] = bt
    b2_ref[...] = bt * bt


def _tc_pre(x, wd, ws, b):
    return pl.pallas_call(
        _pre_body,
        grid=(_GRID,),
        in_specs=[
            pl.BlockSpec((_ROW_BLK, D), lambda i: (i, 0)),
            pl.BlockSpec((D, D), lambda i: (0, 0)),
            pl.BlockSpec((D, D), lambda i: (0, 0)),
            pl.BlockSpec((1, D), lambda i: (0, 0)),
        ],
        out_specs=[
            pl.BlockSpec((_ROW_BLK, D), lambda i: (i, 0)),
            pl.BlockSpec((_ROW_BLK, D), lambda i: (i, 0)),
            pl.BlockSpec((_ROW_BLK, D), lambda i: (i, 0)),
        ],
        out_shape=[jax.ShapeDtypeStruct((N, D), jnp.float32)] * 3,
    )(x, wd, ws, b.reshape(1, D))


def _bnpre_body(h_ref, s_ref, sq_ref, g_ref, be_ref, wd_ref, ws_ref, b_ref,
                hn_ref, a_ref, bt_ref, b2_ref):
    mu = s_ref[...] / N
    var = sq_ref[...] / N - mu * mu
    inv = lax.rsqrt(var + 1e-5) * g_ref[...]
    hn = jnp.maximum((h_ref[...] - mu) * inv + be_ref[...], 0.0)
    hn_ref[...] = hn
    a_ref[...] = jnp.dot(hn, wd_ref[...], preferred_element_type=jnp.float32) + b_ref[...]
    bt = jnp.dot(hn, ws_ref[...], preferred_element_type=jnp.float32)
    bt_ref[...] = bt
    b2_ref[...] = bt * bt


def _tc_bn_pre(h_raw, s, sq, gamma, beta, wd, ws, b):
    return pl.pallas_call(
        _bnpre_body,
        grid=(_GRID,),
        in_specs=[
            pl.BlockSpec((_ROW_BLK, D), lambda i: (i, 0)),
            pl.BlockSpec((1, D), lambda i: (0, 0)),
            pl.BlockSpec((1, D), lambda i: (0, 0)),
            pl.BlockSpec((1, D), lambda i: (0, 0)),
            pl.BlockSpec((1, D), lambda i: (0, 0)),
            pl.BlockSpec((D, D), lambda i: (0, 0)),
            pl.BlockSpec((D, D), lambda i: (0, 0)),
            pl.BlockSpec((1, D), lambda i: (0, 0)),
        ],
        out_specs=[pl.BlockSpec((_ROW_BLK, D), lambda i: (i, 0))] * 4,
        out_shape=[jax.ShapeDtypeStruct((N, D), jnp.float32)] * 4,
    )(h_raw, s, sq, gamma.reshape(1, D), beta.reshape(1, D), wd, ws, b.reshape(1, D))


def _post_body(x_ref, a_ref, s_ref, sq_ref, mn_ref, mx_ref, cnt_ref,
               pw_ref, pb_ref, lw_ref, lb_ref,
               y_ref, ps_ref, psq_ref):
    i = pl.program_id(0)
    count = cnt_ref[:, 0:1]
    cnt = jnp.maximum(count, 1.0)
    has = count > 0.0
    a = a_ref[...]
    s = s_ref[...]
    mean = (count * a + s) / cnt
    mn = jnp.where(has, a + mn_ref[...], 0.0)
    mx = jnp.where(has, a + mx_ref[...], 0.0)
    sc = s / cnt
    std = jnp.sqrt(jnp.maximum(sq_ref[...] / cnt - sc * sc, 0.0) + 1e-5)
    aggr = jnp.concatenate([mean, mn, mx, std], axis=-1)
    lg = jnp.log(cnt + 1.0)
    amp = lg * (1.0 / _AVG_DEG_LOG)
    att = _AVG_DEG_LOG / lg
    p0 = pw_ref[0:D, :]
    p1 = pw_ref[D:5 * D, :]
    p2 = pw_ref[5 * D:9 * D, :]
    p3 = pw_ref[9 * D:13 * D, :]
    y = (jnp.dot(x_ref[...], p0, preferred_element_type=jnp.float32)
         + jnp.dot(aggr, p1, preferred_element_type=jnp.float32)
         + amp * jnp.dot(aggr, p2, preferred_element_type=jnp.float32)
         + att * jnp.dot(aggr, p3, preferred_element_type=jnp.float32)
         + pb_ref[...])
    y = jnp.dot(y, lw_ref[...], preferred_element_type=jnp.float32) + lb_ref[...]
    y_ref[...] = y

    @pl.when(i == 0)
    def _():
        ps_ref[...] = jnp.zeros_like(ps_ref)
        psq_ref[...] = jnp.zeros_like(psq_ref)

    ps_ref[...] += jnp.sum(y, axis=0, keepdims=True)
    psq_ref[...] += jnp.sum(y * y, axis=0, keepdims=True)


def _tc_post(x, a, s, sq, mn, mx, cnt16, post_w, post_b, lin_w, lin_b):
    return pl.pallas_call(
        _post_body,
        grid=(_GRID,),
        in_specs=[
            pl.BlockSpec((_ROW_BLK, D), lambda i: (i, 0)),
            pl.BlockSpec((_ROW_BLK, D), lambda i: (i, 0)),
            pl.BlockSpec((_ROW_BLK, D), lambda i: (i, 0)),
            pl.BlockSpec((_ROW_BLK, D), lambda i: (i, 0)),
            pl.BlockSpec((_ROW_BLK, D), lambda i: (i, 0)),
            pl.BlockSpec((_ROW_BLK, D), lambda i: (i, 0)),
            pl.BlockSpec((_ROW_BLK, 16), lambda i: (i, 0)),
            pl.BlockSpec((13 * D, D), lambda i: (0, 0)),
            pl.BlockSpec((1, D), lambda i: (0, 0)),
            pl.BlockSpec((D, D), lambda i: (0, 0)),
            pl.BlockSpec((1, D), lambda i: (0, 0)),
        ],
        out_specs=[
            pl.BlockSpec((_ROW_BLK, D), lambda i: (i, 0)),
            pl.BlockSpec((1, D), lambda i: (0, 0)),
            pl.BlockSpec((1, D), lambda i: (0, 0)),
        ],
        out_shape=[
            jax.ShapeDtypeStruct((N, D), jnp.float32),
            jax.ShapeDtypeStruct((1, D), jnp.float32),
            jax.ShapeDtypeStruct((1, D), jnp.float32),
        ],
    )(x, a, s, sq, mn, mx, cnt16, post_w, post_b.reshape(1, D),
      lin_w, lin_b.reshape(1, D))


def _final_body(h_ref, s_ref, sq_ref, g_ref, be_ref, x_ref, o_ref):
    mu = s_ref[...] / N
    var = sq_ref[...] / N - mu * mu
    inv = lax.rsqrt(var + 1e-5) * g_ref[...]
    hn = jnp.maximum((h_ref[...] - mu) * inv + be_ref[...], 0.0)
    o_ref[...] = (hn + x_ref[...]) * np.float32(1.0 / math.sqrt(2.0))


def _tc_final(h_raw, s, sq, gamma, beta, x):
    return pl.pallas_call(
        _final_body,
        grid=(_GRID,),
        in_specs=[
            pl.BlockSpec((_ROW_BLK, D), lambda i: (i, 0)),
            pl.BlockSpec((1, D), lambda i: (0, 0)),
            pl.BlockSpec((1, D), lambda i: (0, 0)),
            pl.BlockSpec((1, D), lambda i: (0, 0)),
            pl.BlockSpec((1, D), lambda i: (0, 0)),
            pl.BlockSpec((_ROW_BLK, D), lambda i: (i, 0)),
        ],
        out_specs=pl.BlockSpec((_ROW_BLK, D), lambda i: (i, 0)),
        out_shape=jax.ShapeDtypeStruct((N, D), jnp.float32),
    )(h_raw, s, sq, gamma.reshape(1, D), beta.reshape(1, D), x)


_NT = 32
_NPT = 320
_NPAD = _NT * _NPT
_TRASH = _NPT
_C1 = 80
_EPT = E // 16
_C2 = 4000
_GB = 32
_NBMAX = _C2 // _GB + 1


def _sc_p1_body(tb, srch, dsth, ssq_out, stage, dstb, srcb, sacc):
    c = lax.axis_index("c")
    s = lax.axis_index("s")

    fzero = jnp.zeros((16,), jnp.float32)

    def _zstage(i, carry):
        stage[i >> 3, pl.ds((i & 7) * 16, 16)] = fzero
        return carry
    lax.fori_loop(0, 104 * 8, _zstage, 0)

    base = s * 624
    def _zsp(j, carry):
        pltpu.sync_copy(stage.at[pl.ds(0, 104)], sacc.at[pl.ds(base + j * 104, 104)])
        return carry
    lax.fori_loop(0, 6, _zsp, 0)

    @pl.when(s == 0)
    def _():
        pltpu.sync_copy(stage.at[pl.ds(0, 16)], sacc.at[pl.ds(9984, 16)])

    plsc.subcore_barrier()

    cN = c * N
    ebase = s * _EPT
    def _p1(ch, carry):
        e0 = ebase + ch * _C1
        pltpu.sync_copy(dsth.at[pl.ds(e0, _C1)], dstb)
        pltpu.sync_copy(srch.at[pl.ds(e0, _C1)], srcb)
        for j in range(_C1 // 16):
            srcb[pl.ds(j * 16, 16)] = srcb[pl.ds(j * 16, 16)] + cN
        stg = stage.at[pl.ds(0, _C1)]
        pltpu.sync_copy(tb.at[srcb], stg)
        pltpu.sync_copy(stg, sacc.at[dstb], add=True)
        return carry
    lax.fori_loop(0, _EPT // _C1, _p1, 0)

    plsc.subcore_barrier()

    ob = cN + base
    pltpu.sync_copy(sacc.at[pl.ds(base, 624)], ssq_out.at[pl.ds(ob, 624)])

    @pl.when(s == 0)
    def _():
        pltpu.sync_copy(sacc.at[pl.ds(9984, 16)], ssq_out.at[pl.ds(cN + 9984, 16)])


def _sc_p2_body(bt, srch, dsth, cnt_out, mn_out, mx_out, lists_out,
                dchunk, schunk, mbuf,
                gidx, relb, brows, accmn, accmx, ccnt, cstage):
    c = lax.axis_index("c")
    s = lax.axis_index("s")
    wid = c * 16 + s
    i32 = jnp.int32
    fzero = jnp.zeros((16,), jnp.float32)
    fone = jnp.ones((16,), jnp.float32)
    zi = jnp.zeros((16,), i32)

    fmaxv = jnp.full((16,), 3.4028235e38, jnp.float32)
    fminv = -fmaxv

    def _iacc(r, carry):
        for k in range(8):
            accmn[r, pl.ds(k * 16, 16)] = fmaxv
            accmx[r, pl.ds(k * 16, 16)] = fminv
        ccnt[pl.ds(r * 16, 16)] = fzero
        return carry
    lax.fori_loop(0, _NPT + 1, _iacc, 0)

    lo = wid * _NPT
    trashpack = zi + (_NPT * 16384)

    def _p2(ch, carry):
        pltpu.sync_copy(dsth.at[pl.ds(ch * _C2, _C2)], dchunk)
        pltpu.sync_copy(srch.at[pl.ds(ch * _C2, _C2)], schunk)

        def _scan(g, mcnt):
            dv = dchunk[pl.ds(g * 16, 16)]
            sv = schunk[pl.ds(g * 16, 16)]
            rel = dv - lo
            m = (rel >= 0) & (rel < _NPT)
            mi = jnp.where(m, 1, 0)
            pv = jnp.where(m, rel * 16384 + sv, _NPT * 16384)
            for ln in range(16):
                mbuf[pl.ds(mcnt, 16)] = zi + pv[ln]
                mcnt = mcnt + mi[ln]
            return mcnt
        mcnt = lax.fori_loop(0, _C2 // 16, _scan, jnp.int32(0))

        mbuf[pl.ds(mcnt, 16)] = trashpack
        mbuf[pl.ds(mcnt + 16, 16)] = trashpack
        mbuf[pl.ds(_C2 + 32, 16)] = zi + mcnt
        pltpu.sync_copy(mbuf, lists_out.at[pl.ds((wid * (E // _C2) + ch) * (_C2 + 48), _C2 + 48)])

        def _batch(b, carry2):
            @pl.when(b * _GB < mcnt)
            def _():
                for grp in range(2):
                    pvv = mbuf[pl.ds(b * _GB + grp * 16, 16)]
                    gidx[pl.ds(grp * 16, 16)] = pvv & 16383
                    relb[pl.ds(grp * 16, 16)] = lax.shift_right_arithmetic(pvv, 14)
                pltpu.sync_copy(bt.at[gidx], brows)
                for grp in range(2):
                    relv = relb[pl.ds(grp * 16, 16)] + 0
                    for ln in range(16):
                        r = relv[ln]
                        l = grp * 16 + ln
                        ccnt[pl.ds(r * 16, 16)] = ccnt[pl.ds(r * 16, 16)] + fone
                        for k in range(8):
                            seg = brows[l, pl.ds(k * 16, 16)]
                            accmn[r, pl.ds(k * 16, 16)] = jnp.minimum(
                                accmn[r, pl.ds(k * 16, 16)], seg)
                            accmx[r, pl.ds(k * 16, 16)] = jnp.maximum(
                                accmx[r, pl.ds(k * 16, 16)], seg)
            return carry2
        lax.fori_loop(0, _NBMAX, _batch, 0)
        return carry
    lax.fori_loop(0, E // _C2, _p2, 0)

    pltpu.sync_copy(accmn.at[pl.ds(0, _NPT)], mn_out.at[pl.ds(lo, _NPT)])
    pltpu.sync_copy(accmx.at[pl.ds(0, _NPT)], mx_out.at[pl.ds(lo, _NPT)])

    def _rp(j, carry):
        cstage[j >> 3, pl.ds((j & 7) * 16, 16)] = ccnt[pl.ds(j * 16, 16)] + 0.0
        return carry
    lax.fori_loop(0, _NPT, _rp, 0)
    pltpu.sync_copy(cstage.at[pl.ds(0, 40)], cnt_out.at[pl.ds(wid * 40, 40)])


def _sc_p2b_body(bt, lists_in, mn_out, mx_out,
                 mbuf, gidx, relb, brows, accmn, accmx):
    c = lax.axis_index("c")
    s = lax.axis_index("s")
    wid = c * 16 + s
    i32 = jnp.int32
    zi = jnp.zeros((16,), i32)

    fmaxv = jnp.full((16,), 3.4028235e38, jnp.float32)
    fminv = -fmaxv

    def _iacc(r, carry):
        for k in range(8):
            accmn[r, pl.ds(k * 16, 16)] = fmaxv
            accmx[r, pl.ds(k * 16, 16)] = fminv
        return carry
    lax.fori_loop(0, _NPT + 1, _iacc, 0)

    lo = wid * _NPT

    def _p2(ch, carry):
        pltpu.sync_copy(lists_in.at[pl.ds((wid * (E // _C2) + ch) * (_C2 + 48), _C2 + 48)], mbuf)
        mcv = mbuf[pl.ds(_C2 + 32, 16)] + 0
        mcnt = mcv[0]

        def _batch(b, carry2):
            @pl.when(b * _GB < mcnt)
            def _():
                for grp in range(2):
                    pvv = mbuf[pl.ds(b * _GB + grp * 16, 16)]
                    gidx[pl.ds(grp * 16, 16)] = pvv & 16383
                    relb[pl.ds(grp * 16, 16)] = lax.shift_right_arithmetic(pvv, 14)
                pltpu.sync_copy(bt.at[gidx], brows)
                for grp in range(2):
                    relv = relb[pl.ds(grp * 16, 16)] + 0
                    for ln in range(16):
                        r = relv[ln]
                        l = grp * 16 + ln
                        for k in range(8):
                            seg = brows[l, pl.ds(k * 16, 16)]
                            accmn[r, pl.ds(k * 16, 16)] = jnp.minimum(
                                accmn[r, pl.ds(k * 16, 16)], seg)
                            accmx[r, pl.ds(k * 16, 16)] = jnp.maximum(
                                accmx[r, pl.ds(k * 16, 16)], seg)
            return carry2
        lax.fori_loop(0, _NBMAX, _batch, 0)
        return carry
    lax.fori_loop(0, E // _C2, _p2, 0)

    pltpu.sync_copy(accmn.at[pl.ds(0, _NPT)], mn_out.at[pl.ds(lo, _NPT)])
    pltpu.sync_copy(accmx.at[pl.ds(0, _NPT)], mx_out.at[pl.ds(lo, _NPT)])


def _sc_segment_stats(bt, b2, src, dst, lists=None):
    mesh = plsc.VectorSubcoreMesh(core_axis_name="c", subcore_axis_name="s")
    f32 = jnp.float32
    k1 = pl.kernel(
        _sc_p1_body,
        out_type=[
            jax.ShapeDtypeStruct((2 * N, D), f32),
        ],
        mesh=mesh,
        scratch_types=[
            pltpu.VMEM((104, D), f32),
            pltpu.VMEM((_C1,), jnp.int32),
            pltpu.VMEM((_C1,), jnp.int32),
            pltpu.VMEM_SHARED((N, D), f32),
        ],
    )
    tb = jnp.concatenate([bt, b2], axis=0)
    ssq = k1(tb, src, dst)[0]
    s, sq = ssq[:N], ssq[N:]
    k2 = pl.kernel(
        _sc_p2_body,
        out_type=[
            jax.ShapeDtypeStruct((_NPAD // 8, D), f32),
            jax.ShapeDtypeStruct((_NPAD, D), f32),
            jax.ShapeDtypeStruct((_NPAD, D), f32),
            jax.ShapeDtypeStruct((_NT * (E // _C2) * (_C2 + 48),), jnp.int32),
        ],
        mesh=mesh,
        scratch_types=[
            pltpu.VMEM((_C2,), jnp.int32),
            pltpu.VMEM((_C2,), jnp.int32),
            pltpu.VMEM((_C2 + 48,), jnp.int32),
            pltpu.VMEM((32,), jnp.int32),
            pltpu.VMEM((32,), jnp.int32),
            pltpu.VMEM((32, D), f32),
            pltpu.VMEM((_NPT + 1, D), f32),
            pltpu.VMEM((_NPT + 1, D), f32),
            pltpu.VMEM(((_NPT + 1) * 16,), f32),
            pltpu.VMEM((40, D), f32),
        ],
    )
    if lists is None:
        cntp, mn, mx, lists_o = k2(bt, src, dst)
        cnt16 = cntp.reshape(_NPAD, 16)[:N]
        return s, sq, mn[:N], mx[:N], cnt16, lists_o
    k2b = pl.kernel(
        _sc_p2b_body,
        out_type=[
            jax.ShapeDtypeStruct((_NPAD, D), f32),
            jax.ShapeDtypeStruct((_NPAD, D), f32),
        ],
        mesh=mesh,
        scratch_types=[
            pltpu.VMEM((_C2 + 48,), jnp.int32),
            pltpu.VMEM((32,), jnp.int32),
            pltpu.VMEM((32,), jnp.int32),
            pltpu.VMEM((32, D), f32),
            pltpu.VMEM((_NPT + 1, D), f32),
            pltpu.VMEM((_NPT + 1, D), f32),
        ],
    )
    mn, mx = k2b(bt, lists)
    return s, sq, mn[:N], mx[:N], None, None


def kernel(x, edge_index, pre_W1, pre_b1, post_W1, post_b1, lin_W1, lin_b1,
           pre_W2, pre_b2, post_W2, post_b2, lin_W2, lin_b2,
           bn1_gamma, bn1_beta, bn2_gamma, bn2_beta):
    src = edge_index[0]
    dst = edge_index[1]

    a1, bt1, b21 = _tc_pre(x, pre_W1[:D], pre_W1[D:], pre_b1)
    s1, sq1, mn1, mx1, cnt16, lists = _sc_segment_stats(bt1, b21, src, dst)
    h1_raw, cs1, csq1 = _tc_post(x, a1, s1, sq1, mn1, mx1, cnt16,
                                 post_W1, post_b1, lin_W1, lin_b1)

    h1, a2, bt2, b22 = _tc_bn_pre(h1_raw, cs1, csq1, bn1_gamma, bn1_beta,
                                  pre_W2[:D], pre_W2[D:], pre_b2)
    s2, sq2, mn2, mx2, _, _ = _sc_segment_stats(bt2, b22, src, dst, lists=lists)
    h2_raw, cs2, csq2 = _tc_post(h1, a2, s2, sq2, mn2, mx2, cnt16,
                                 post_W2, post_b2, lin_W2, lin_b2)

    return _tc_final(h2_raw, cs2, csq2, bn2_gamma, bn2_beta, x)
